# baseline (speedup 1.0000x reference)
import functools

import jax
import jax.numpy as jnp
from jax import lax
from jax.experimental import pallas as pl
from jax.experimental.pallas import tpu as pltpu
from jax.experimental.pallas import tpu_sc as plsc

_INFO = plsc.get_sparse_core_info()
_NC = _INFO.num_cores
_NS = _INFO.num_subcores
_NW = _NC * _NS

_D = 768
_B = 4 * 8192
_PER_W = _B // _NW
_CHUNK = 32
_NCHUNK = _PER_W // _CHUNK
_NBUF = 4


def _sc_gather(xf, W):
    mesh = plsc.VectorSubcoreMesh(core_axis_name="c", subcore_axis_name="s")

    @functools.partial(
        pl.kernel,
        out_type=jax.ShapeDtypeStruct((_B, _D), jnp.float32),
        mesh=mesh,
        scratch_types=[
            pltpu.VMEM((_NCHUNK, _CHUNK), jnp.int32),
            pltpu.VMEM((_NBUF, _CHUNK, _D), jnp.float32),
            pltpu.SemaphoreType.DMA,
            pltpu.SemaphoreType.DMA,
        ],
    )
    def k(x_hbm, w_hbm, out_hbm, idx_v, rows_v, gsem, ssem):
        wid = lax.axis_index("s") * _NC + lax.axis_index("c")
        base = wid * _PER_W
        pltpu.sync_copy(x_hbm.at[wid], idx_v)

        def gather(g):
            return pltpu.async_copy(
                w_hbm.at[idx_v.at[g]], rows_v.at[g % _NBUF], gsem)

        def scatter(g):
            return pltpu.async_copy(
                rows_v.at[g % _NBUF],
                out_hbm.at[pl.ds(base + g * _CHUNK, _CHUNK)], ssem)

        gathers = [gather(0)]
        scatters = []
        for g in range(_NCHUNK):
            if g >= _NBUF - 1:
                scatters[g - (_NBUF - 1)].wait()
            if g + 1 < _NCHUNK:
                gathers.append(gather(g + 1))
            gathers[g].wait()
            scatters.append(scatter(g))
        for s in scatters[-(_NBUF - 1):]:
            s.wait()

    return k(xf, W)


def kernel(x, W):
    xf = x.reshape(_NW, _NCHUNK, _CHUNK).astype(jnp.int32)
    out = _sc_gather(xf, W)
    return out.reshape(x.shape[0], x.shape[1], _D)


# speedup vs baseline: 1.0037x; 1.0037x over previous
import functools

import jax
import jax.numpy as jnp
from jax import lax
from jax.experimental import pallas as pl
from jax.experimental.pallas import tpu as pltpu
from jax.experimental.pallas import tpu_sc as plsc

_INFO = plsc.get_sparse_core_info()
_NC = _INFO.num_cores
_NS = _INFO.num_subcores
_NW = _NC * _NS

_D = 768
_B = 4 * 8192
_PER_W = _B // _NW
_CHUNK = 64
_NCHUNK = _PER_W // _CHUNK
_NBUF = 2


def _sc_gather(xf, W):
    mesh = plsc.VectorSubcoreMesh(core_axis_name="c", subcore_axis_name="s")

    @functools.partial(
        pl.kernel,
        out_type=jax.ShapeDtypeStruct((_B, _D), jnp.float32),
        mesh=mesh,
        scratch_types=[
            pltpu.VMEM((_NCHUNK, _CHUNK), jnp.int32),
            pltpu.VMEM((_NBUF, _CHUNK, _D), jnp.float32),
            pltpu.SemaphoreType.DMA,
            pltpu.SemaphoreType.DMA,
        ],
    )
    def k(x_hbm, w_hbm, out_hbm, idx_v, rows_v, gsem, ssem):
        wid = lax.axis_index("s") * _NC + lax.axis_index("c")
        base = wid * _PER_W
        pltpu.sync_copy(x_hbm.at[wid], idx_v)

        def gather(g):
            return pltpu.async_copy(
                w_hbm.at[idx_v.at[g]], rows_v.at[g % _NBUF], gsem)

        def scatter(g):
            return pltpu.async_copy(
                rows_v.at[g % _NBUF],
                out_hbm.at[pl.ds(base + g * _CHUNK, _CHUNK)], ssem)

        gathers = [gather(0)]
        scatters = []
        for g in range(_NCHUNK):
            if g >= _NBUF - 1 and scatters:
                scatters[g - (_NBUF - 1)].wait()
            if g + 1 < _NCHUNK:
                gathers.append(gather(g + 1))
            gathers[g].wait()
            scatters.append(scatter(g))
        for s in scatters[-(_NBUF - 1):]:
            s.wait()

    return k(xf, W)


def kernel(x, W):
    xf = x.reshape(_NW, _NCHUNK, _CHUNK).astype(jnp.int32)
    out = _sc_gather(xf, W)
    return out.reshape(x.shape[0], x.shape[1], _D)
